# Initial kernel scaffold; baseline (speedup 1.0000x reference)
import jax
import jax.numpy as jnp
from jax.experimental import pallas as pl


def kernel(x, edge_index, W_in, b_in, gW0, gb0, gW1, gb1, gW2, gb2, lg0, lb0, lg1, lb1, lg2, lb2, att_W, att_asrc, att_adst, att_b, W_ao, b_ao, W_out, b_out):
    raise NotImplementedError("write your pallas kernel here")


# speedup vs baseline: 1.0124x; 1.0124x over previous
import functools
import jax, jax.numpy as jnp
from jax.experimental import pallas as pl

N=10000; E=320000; H=128; HEADS=4

def _mm_kernel(x_ref, w_ref, b_ref, o_ref, *, act):
    y = jnp.dot(x_ref[...], w_ref[...], preferred_element_type=jnp.float32)
    y = y + b_ref[...]
    if act == "relu":
        y = jnp.maximum(y, 0.0)
    o_ref[...] = y

def _mm(x, W, b, act=None, bm=512):
    M, K = x.shape
    _, Nw = W.shape
    return pl.pallas_call(
        functools.partial(_mm_kernel, act=act),
        grid=(M // bm,),
        in_specs=[
            pl.BlockSpec((bm, K), lambda i: (i, 0)),
            pl.BlockSpec((K, Nw), lambda i: (0, 0)),
            pl.BlockSpec((1, Nw), lambda i: (0, 0)),
        ],
        out_specs=pl.BlockSpec((bm, Nw), lambda i: (i, 0)),
        out_shape=jax.ShapeDtypeStruct((M, Nw), jnp.float32),
    )(x, W, b.reshape(1, Nw))

def _ln(x, g, b):
    m = jnp.mean(x, axis=-1, keepdims=True)
    v = jnp.var(x, axis=-1, keepdims=True)
    return (x - m) / jnp.sqrt(v + 1e-5) * g + b

def _gcn(x, src, dst, W, b):
    loop = jnp.arange(N, dtype=src.dtype)
    s = jnp.concatenate([src, loop]); d = jnp.concatenate([dst, loop])
    xw = x @ W
    deg = jax.ops.segment_sum(jnp.ones_like(d, dtype=xw.dtype), d, num_segments=N)
    dinv = jnp.where(deg > 0, 1.0 / jnp.sqrt(deg), 0.0)
    norm = dinv[s] * dinv[d]
    out = jax.ops.segment_sum(xw[s] * norm[:, None], d, num_segments=N)
    return out + b

def _gat(x, src, dst, W, a_src, a_dst, bias):
    loop = jnp.arange(N, dtype=src.dtype)
    s = jnp.concatenate([src, loop]); d = jnp.concatenate([dst, loop])
    xw = (x @ W).reshape(N, HEADS, H)
    al_s = jnp.sum(xw * a_src, axis=-1); al_d = jnp.sum(xw * a_dst, axis=-1)
    e = jax.nn.leaky_relu(al_s[s] + al_d[d], 0.2)
    mx = jax.ops.segment_max(e, d, num_segments=N)
    mx = jnp.where(jnp.isfinite(mx), mx, 0.0)
    ex = jnp.exp(e - mx[d])
    den = jax.ops.segment_sum(ex, d, num_segments=N)
    alpha = ex / (den[d] + 1e-16)
    out = jax.ops.segment_sum(xw[s] * alpha[:, :, None], d, num_segments=N)
    return out.reshape(N, HEADS * H) + bias

def kernel(x, edge_index, W_in, b_in, gW0, gb0, gW1, gb1, gW2, gb2, lg0, lb0, lg1, lb1, lg2, lb2, att_W, att_asrc, att_adst, att_b, W_ao, b_ao, W_out, b_out):
    src = edge_index[0]; dst = edge_index[1]
    xp = jnp.concatenate([x, jnp.zeros((240, x.shape[1]), x.dtype)], axis=0)
    h = _mm(xp, W_in, b_in, act="relu")[:N]
    params = [(gW0, gb0, lg0, lb0), (gW1, gb1, lg1, lb1), (gW2, gb2, lg2, lb2)]
    for i, (gW, gb, lg, lb) in enumerate(params):
        g = _gcn(h, src, dst, gW, gb)
        g = jax.nn.relu(_ln(g, lg, lb))
        h = h + g if i > 0 else g
    a = _gat(h, src, dst, att_W, att_asrc, att_adst, att_b)
    h = jax.nn.relu(a @ W_ao + b_ao)
    return h @ W_out + b_out
